# Initial kernel scaffold; baseline (speedup 1.0000x reference)
import jax
import jax.numpy as jnp
from jax.experimental import pallas as pl


def kernel(x, edge_index, batch, params):
    raise NotImplementedError("write your pallas kernel here")


# speedup vs baseline: 1.4182x; 1.4182x over previous
import functools

import jax
import jax.numpy as jnp
from jax import lax
from jax.experimental import pallas as pl
from jax.experimental.pallas import tpu as pltpu
from jax.experimental.pallas import tpu_sc as plsc

N = 10000
D = 128
G = 512

NC = 2
NS = 16
NW = NC * NS
LANES = 16
SCAN_CHUNK = 1024
FLUSH = 128
CAP = 2 * FLUSH


@functools.partial(jax.jit, static_argnames=("n_outer", "s_pad", "d"))
def _sc_segment_sum(table, src, dst, *, n_outer, s_pad, d):
    mesh = plsc.VectorSubcoreMesh(core_axis_name="c", subcore_axis_name="s")
    rpw = s_pad // NW
    kd = d // LANES

    @functools.partial(
        pl.kernel,
        out_type=jax.ShapeDtypeStruct((s_pad, d), jnp.float32),
        mesh=mesh,
        compiler_params=pltpu.CompilerParams(needs_layout_passes=False),
        scratch_types=[
            pltpu.VMEM((SCAN_CHUNK,), jnp.int32),
            pltpu.VMEM((SCAN_CHUNK,), jnp.int32),
            pltpu.VMEM((CAP,), jnp.int32),
            pltpu.VMEM((CAP,), jnp.int32),
            pltpu.VMEM((FLUSH, d), jnp.float32),
            pltpu.VMEM((rpw + 1, d), jnp.float32),
            pltpu.SemaphoreType.DMA,
        ],
    )
    def k(table_hbm, src_hbm, dst_hbm, out_hbm,
          src_c, dst_c, pend_src, pend_ldst, rows_v, acc, sem):
        c = lax.axis_index("c")
        s = lax.axis_index("s")
        w = s * NC + c
        lo = w * rpw

        zi32 = jnp.zeros((LANES,), jnp.int32)
        zf32 = jnp.zeros((LANES,), jnp.float32)

        def zrow(i, _):
            for kk in range(kd):
                acc[i, pl.ds(kk * LANES, LANES)] = zf32
            return ()

        lax.fori_loop(0, rpw + 1, zrow, ())
        for i in range(CAP // LANES):
            pend_src[pl.ds(i * LANES, LANES)] = zi32
            pend_ldst[pl.ds(i * LANES, LANES)] = zi32

        def accumulate():
            pltpu.async_copy(table_hbm.at[pend_src.at[pl.ds(0, FLUSH)]],
                             rows_v, sem).wait()

            def grp_body(gj, _):
                ldvec = pend_ldst[pl.ds(gj * LANES, LANES)]
                for jj in range(LANES):
                    ldst = ldvec[jj]
                    for kk in range(kd):
                        x = rows_v[gj * LANES + jj, pl.ds(kk * LANES, LANES)]
                        plsc.addupdate(
                            acc.at[ldst, pl.ds(kk * LANES, LANES)], x)
                return ()

            lax.fori_loop(0, FLUSH // LANES, grp_body, ())

        def shift_pending():
            for t in range((CAP - FLUSH) // LANES):
                pend_src[pl.ds(t * LANES, LANES)] = (
                    pend_src[pl.ds(FLUSH + t * LANES, LANES)])
                pend_ldst[pl.ds(t * LANES, LANES)] = (
                    pend_ldst[pl.ds(FLUSH + t * LANES, LANES)])

        def group(g, carry):
            blk_base, cnt = carry
            off = blk_base + g * LANES
            dv = dst_c[pl.ds(off, LANES)]
            sv = src_c[pl.ds(off, LANES)]
            ld = dv - lo
            m = (ld >= 0) & (ld < rpw)
            cs = plsc.cumsum(m.astype(jnp.int32))
            pos = cs + (cnt - 1)
            plsc.store_scatter(pend_src, [pos], sv, mask=m)
            plsc.store_scatter(pend_ldst, [pos], ld, mask=m)
            return (blk_base, cnt + jnp.max(cs))

        def block(b, cnt):
            _, cnt = lax.fori_loop(0, FLUSH // LANES, group,
                                   (b * FLUSH, cnt))
            do = cnt >= FLUSH

            @pl.when(do)
            def _():
                accumulate()
                shift_pending()

            return jnp.where(do, cnt - FLUSH, cnt)

        def outer(o, cnt):
            oe = lax.rem(o + w * (n_outer // NW), n_outer)
            base = oe * SCAN_CHUNK
            pltpu.sync_copy(src_hbm.at[pl.ds(base, SCAN_CHUNK)], src_c)
            pltpu.sync_copy(dst_hbm.at[pl.ds(base, SCAN_CHUNK)], dst_c)
            return lax.fori_loop(0, SCAN_CHUNK // FLUSH, block, cnt)

        cnt = lax.fori_loop(0, n_outer, outer, jnp.int32(0))

        lane = lax.iota(jnp.int32, LANES)

        @pl.when(cnt > 0)
        def _():
            for g in range(FLUSH // LANES):
                v = pend_ldst[pl.ds(g * LANES, LANES)]
                pend_ldst[pl.ds(g * LANES, LANES)] = jnp.where(
                    lane + (g * LANES) < cnt, v, rpw)
            accumulate()

        pltpu.sync_copy(acc.at[pl.ds(0, rpw)], out_hbm.at[pl.ds(lo, rpw)])

    return k(table, src, dst)


def _segment_sum(table, src, dst, s_rows):
    e = src.shape[0]
    d = table.shape[1]
    n_outer = -(-e // SCAN_CHUNK)
    e_pad = n_outer * SCAN_CHUNK
    s_pad = -(-s_rows // (NW * 8)) * (NW * 8)
    if e_pad != e:
        src = jnp.concatenate([src, jnp.zeros((e_pad - e,), jnp.int32)])
        dst = jnp.concatenate([dst, jnp.full((e_pad - e,), s_pad, jnp.int32)])
    return _sc_segment_sum(table, src, dst, n_outer=n_outer, s_pad=s_pad, d=d)


BN_ROWS = 1000


def _lin_body(x_ref, w_ref, b_ref, o_ref):
    o_ref[...] = (
        jnp.dot(x_ref[...], w_ref[...], preferred_element_type=jnp.float32)
        + b_ref[...]
    )


def _initial_linear(x, w, b):
    return pl.pallas_call(
        _lin_body,
        grid=(N // 2000,),
        in_specs=[
            pl.BlockSpec((2000, D), lambda i: (i, 0)),
            pl.BlockSpec((D, D), lambda i: (0, 0)),
            pl.BlockSpec((1, D), lambda i: (0, 0)),
        ],
        out_specs=pl.BlockSpec((2000, D), lambda i: (i, 0)),
        out_shape=jax.ShapeDtypeStruct((N, D), jnp.float32),
    )(x, w, b.reshape(1, D))


def _layer_body(ag_ref, w1_ref, b1_ref, lng_ref, lnb_ref, w2_ref, b2_ref,
                hpre_ref, stats_ref, acc_ref):
    i = pl.program_id(0)
    a = ag_ref[...]
    t = jnp.dot(a, w1_ref[...], preferred_element_type=jnp.float32) + b1_ref[...]
    mu = jnp.mean(t, axis=1, keepdims=True)
    var = jnp.mean((t - mu) ** 2, axis=1, keepdims=True)
    t = (t - mu) * lax.rsqrt(var + 1e-5) * lng_ref[...] + lnb_ref[...]
    t = jnp.maximum(t, 0.0)
    u = jnp.dot(t, w2_ref[...], preferred_element_type=jnp.float32) + b2_ref[...]
    h = jnp.maximum(u + a, 0.0)
    hpre_ref[...] = h

    @pl.when(i == 0)
    def _():
        acc_ref[...] = jnp.zeros_like(acc_ref)

    acc_ref[0:1] += jnp.sum(h, axis=0, keepdims=True)
    acc_ref[1:2] += jnp.sum(h * h, axis=0, keepdims=True)

    @pl.when(i == pl.num_programs(0) - 1)
    def _():
        stats_ref[...] = acc_ref[...]


def _layer_mlp(aggr, p):
    return pl.pallas_call(
        _layer_body,
        grid=(N // BN_ROWS,),
        in_specs=[
            pl.BlockSpec((BN_ROWS, D), lambda i: (i, 0)),
            pl.BlockSpec((D, 2 * D), lambda i: (0, 0)),
            pl.BlockSpec((1, 2 * D), lambda i: (0, 0)),
            pl.BlockSpec((1, 2 * D), lambda i: (0, 0)),
            pl.BlockSpec((1, 2 * D), lambda i: (0, 0)),
            pl.BlockSpec((2 * D, D), lambda i: (0, 0)),
            pl.BlockSpec((1, D), lambda i: (0, 0)),
        ],
        out_specs=[
            pl.BlockSpec((BN_ROWS, D), lambda i: (i, 0)),
            pl.BlockSpec((2, D), lambda i: (0, 0)),
        ],
        out_shape=[
            jax.ShapeDtypeStruct((N, D), jnp.float32),
            jax.ShapeDtypeStruct((2, D), jnp.float32),
        ],
        scratch_shapes=[pltpu.VMEM((2, D), jnp.float32)],
    )(aggr, p["W1"], p["b1"].reshape(1, -1), p["ln_g"].reshape(1, -1),
      p["ln_b"].reshape(1, -1), p["W2"], p["b2"].reshape(1, -1))


def _bn_body(h_ref, stats_ref, g_ref, b_ref, o_ref):
    m = stats_ref[0:1] * (1.0 / N)
    v = stats_ref[1:2] * (1.0 / N) - m * m
    scale = lax.rsqrt(v + 1e-5) * g_ref[...]
    o_ref[...] = (h_ref[...] - m) * scale + b_ref[...]


def _batchnorm(h_pre, stats, g, b):
    return pl.pallas_call(
        _bn_body,
        grid=(N // 2000,),
        in_specs=[
            pl.BlockSpec((2000, D), lambda i: (i, 0)),
            pl.BlockSpec((2, D), lambda i: (0, 0)),
            pl.BlockSpec((1, D), lambda i: (0, 0)),
            pl.BlockSpec((1, D), lambda i: (0, 0)),
        ],
        out_specs=pl.BlockSpec((2000, D), lambda i: (i, 0)),
        out_shape=jax.ShapeDtypeStruct((N, D), jnp.float32),
    )(h_pre, stats, g.reshape(1, D), b.reshape(1, D))


def kernel(x, edge_index, batch, params):
    src = edge_index[0]
    dst = edge_index[1]

    h = _initial_linear(x, params["W_lin"], params["b_lin"])

    xs = []
    for p in params["layers"]:
        aggr = _segment_sum(h, src, dst, N)
        h_pre, stats = _layer_mlp(aggr, p)
        h = _batchnorm(h_pre, stats, p["bn_g"], p["bn_b"])
        xs.append(h)

    xcat = jnp.concatenate(xs, axis=1)
    pool = _segment_sum(xcat, jnp.arange(N, dtype=jnp.int32), batch, G)
    return (pool[:G], xcat)


# speedup vs baseline: 1.8881x; 1.3313x over previous
import functools

import jax
import jax.numpy as jnp
from jax import lax
from jax.experimental import pallas as pl
from jax.experimental.pallas import tpu as pltpu
from jax.experimental.pallas import tpu_sc as plsc

N = 10000
D = 128
G = 512

NC = 2
NS = 16
NW = NC * NS
LANES = 16
SCAN_CHUNK = 4096
FLUSH = 128
CAP = 2 * FLUSH


@functools.partial(jax.jit, static_argnames=("n_outer", "s_pad", "d"))
def _sc_segment_sum(table, src, dst, *, n_outer, s_pad, d):
    mesh = plsc.VectorSubcoreMesh(core_axis_name="c", subcore_axis_name="s")
    rpw = s_pad // NW
    kd = d // LANES

    @functools.partial(
        pl.kernel,
        out_type=jax.ShapeDtypeStruct((s_pad, d), jnp.float32),
        mesh=mesh,
        compiler_params=pltpu.CompilerParams(needs_layout_passes=False),
        scratch_types=[
            pltpu.VMEM((SCAN_CHUNK,), jnp.int32),
            pltpu.VMEM((SCAN_CHUNK,), jnp.int32),
            pltpu.VMEM((CAP,), jnp.int32),
            pltpu.VMEM((CAP,), jnp.int32),
            pltpu.VMEM((FLUSH, d), jnp.float32),
            pltpu.VMEM((rpw + 1, d), jnp.float32),
            pltpu.SemaphoreType.DMA,
        ],
    )
    def k(table_hbm, src_hbm, dst_hbm, out_hbm,
          src_c, dst_c, pend_src, pend_ldst, rows_v, acc, sem):
        c = lax.axis_index("c")
        s = lax.axis_index("s")
        w = s * NC + c
        lo = w * rpw

        zi32 = jnp.zeros((LANES,), jnp.int32)
        zf32 = jnp.zeros((LANES,), jnp.float32)

        def zrow(i, _):
            for kk in range(kd):
                acc[i, pl.ds(kk * LANES, LANES)] = zf32
            return ()

        lax.fori_loop(0, rpw + 1, zrow, ())
        for i in range(CAP // LANES):
            pend_src[pl.ds(i * LANES, LANES)] = zi32
            pend_ldst[pl.ds(i * LANES, LANES)] = zi32

        def accumulate():
            pltpu.async_copy(table_hbm.at[pend_src.at[pl.ds(0, FLUSH)]],
                             rows_v, sem).wait()

            def grp_body(gj, _):
                ldvec = pend_ldst[pl.ds(gj * LANES, LANES)]
                for jj in range(LANES):
                    ldst = ldvec[jj]
                    for kk in range(kd):
                        x = rows_v[gj * LANES + jj, pl.ds(kk * LANES, LANES)]
                        plsc.addupdate(
                            acc.at[ldst, pl.ds(kk * LANES, LANES)], x)
                return ()

            lax.fori_loop(0, FLUSH // LANES, grp_body, ())

        def shift_pending():
            for t in range((CAP - FLUSH) // LANES):
                pend_src[pl.ds(t * LANES, LANES)] = (
                    pend_src[pl.ds(FLUSH + t * LANES, LANES)])
                pend_ldst[pl.ds(t * LANES, LANES)] = (
                    pend_ldst[pl.ds(FLUSH + t * LANES, LANES)])

        def group(g, carry):
            blk_base, cnt = carry
            off = blk_base + g * LANES
            dv = dst_c[pl.ds(off, LANES)]
            sv = src_c[pl.ds(off, LANES)]
            ld = dv - lo
            m = (ld >= 0) & (ld < rpw)
            cs = plsc.cumsum(m.astype(jnp.int32))
            pos = cs + (cnt - 1)
            plsc.store_scatter(pend_src, [pos], sv, mask=m)
            plsc.store_scatter(pend_ldst, [pos], ld, mask=m)
            return (blk_base, cnt + cs[LANES - 1])

        def block(b, cnt):
            _, cnt = lax.fori_loop(0, FLUSH // LANES, group,
                                   (b * FLUSH, cnt), unroll=4)
            do = cnt >= FLUSH

            @pl.when(do)
            def _():
                accumulate()
                shift_pending()

            return jnp.where(do, cnt - FLUSH, cnt)

        def outer(o, cnt):
            oe = lax.rem(o + w * (n_outer // NW), n_outer)
            base = oe * SCAN_CHUNK
            cp1 = pltpu.async_copy(src_hbm.at[pl.ds(base, SCAN_CHUNK)],
                                   src_c, sem)
            cp2 = pltpu.async_copy(dst_hbm.at[pl.ds(base, SCAN_CHUNK)],
                                   dst_c, sem)
            cp1.wait()
            cp2.wait()
            return lax.fori_loop(0, SCAN_CHUNK // FLUSH, block, cnt)

        cnt = lax.fori_loop(0, n_outer, outer, jnp.int32(0))

        lane = lax.iota(jnp.int32, LANES)

        @pl.when(cnt > 0)
        def _():
            for g in range(FLUSH // LANES):
                v = pend_ldst[pl.ds(g * LANES, LANES)]
                pend_ldst[pl.ds(g * LANES, LANES)] = jnp.where(
                    lane + (g * LANES) < cnt, v, rpw)
            accumulate()

        pltpu.sync_copy(acc.at[pl.ds(0, rpw)], out_hbm.at[pl.ds(lo, rpw)])

    return k(table, src, dst)


def _segment_sum(table, src, dst, s_rows):
    e = src.shape[0]
    d = table.shape[1]
    n_outer = -(-e // SCAN_CHUNK)
    e_pad = n_outer * SCAN_CHUNK
    s_pad = -(-s_rows // (NW * 8)) * (NW * 8)
    if e_pad != e:
        src = jnp.concatenate([src, jnp.zeros((e_pad - e,), jnp.int32)])
        dst = jnp.concatenate([dst, jnp.full((e_pad - e,), s_pad, jnp.int32)])
    return _sc_segment_sum(table, src, dst, n_outer=n_outer, s_pad=s_pad, d=d)


BN_ROWS = 1000


def _lin_body(x_ref, w_ref, b_ref, o_ref):
    o_ref[...] = (
        jnp.dot(x_ref[...], w_ref[...], preferred_element_type=jnp.float32)
        + b_ref[...]
    )


def _initial_linear(x, w, b):
    return pl.pallas_call(
        _lin_body,
        grid=(N // 2000,),
        in_specs=[
            pl.BlockSpec((2000, D), lambda i: (i, 0)),
            pl.BlockSpec((D, D), lambda i: (0, 0)),
            pl.BlockSpec((1, D), lambda i: (0, 0)),
        ],
        out_specs=pl.BlockSpec((2000, D), lambda i: (i, 0)),
        out_shape=jax.ShapeDtypeStruct((N, D), jnp.float32),
    )(x, w, b.reshape(1, D))


def _layer_body(ag_ref, w1_ref, b1_ref, lng_ref, lnb_ref, w2_ref, b2_ref,
                hpre_ref, stats_ref, acc_ref):
    i = pl.program_id(0)
    a = ag_ref[...]
    t = jnp.dot(a, w1_ref[...], preferred_element_type=jnp.float32) + b1_ref[...]
    mu = jnp.mean(t, axis=1, keepdims=True)
    var = jnp.mean((t - mu) ** 2, axis=1, keepdims=True)
    t = (t - mu) * lax.rsqrt(var + 1e-5) * lng_ref[...] + lnb_ref[...]
    t = jnp.maximum(t, 0.0)
    u = jnp.dot(t, w2_ref[...], preferred_element_type=jnp.float32) + b2_ref[...]
    h = jnp.maximum(u + a, 0.0)
    hpre_ref[...] = h

    @pl.when(i == 0)
    def _():
        acc_ref[...] = jnp.zeros_like(acc_ref)

    acc_ref[0:1] += jnp.sum(h, axis=0, keepdims=True)
    acc_ref[1:2] += jnp.sum(h * h, axis=0, keepdims=True)

    @pl.when(i == pl.num_programs(0) - 1)
    def _():
        stats_ref[...] = acc_ref[...]


def _layer_mlp(aggr, p):
    return pl.pallas_call(
        _layer_body,
        grid=(N // BN_ROWS,),
        in_specs=[
            pl.BlockSpec((BN_ROWS, D), lambda i: (i, 0)),
            pl.BlockSpec((D, 2 * D), lambda i: (0, 0)),
            pl.BlockSpec((1, 2 * D), lambda i: (0, 0)),
            pl.BlockSpec((1, 2 * D), lambda i: (0, 0)),
            pl.BlockSpec((1, 2 * D), lambda i: (0, 0)),
            pl.BlockSpec((2 * D, D), lambda i: (0, 0)),
            pl.BlockSpec((1, D), lambda i: (0, 0)),
        ],
        out_specs=[
            pl.BlockSpec((BN_ROWS, D), lambda i: (i, 0)),
            pl.BlockSpec((2, D), lambda i: (0, 0)),
        ],
        out_shape=[
            jax.ShapeDtypeStruct((N, D), jnp.float32),
            jax.ShapeDtypeStruct((2, D), jnp.float32),
        ],
        scratch_shapes=[pltpu.VMEM((2, D), jnp.float32)],
    )(aggr, p["W1"], p["b1"].reshape(1, -1), p["ln_g"].reshape(1, -1),
      p["ln_b"].reshape(1, -1), p["W2"], p["b2"].reshape(1, -1))


def _bn_body(h_ref, stats_ref, g_ref, b_ref, o_ref):
    m = stats_ref[0:1] * (1.0 / N)
    v = stats_ref[1:2] * (1.0 / N) - m * m
    scale = lax.rsqrt(v + 1e-5) * g_ref[...]
    o_ref[...] = (h_ref[...] - m) * scale + b_ref[...]


def _batchnorm(h_pre, stats, g, b):
    return pl.pallas_call(
        _bn_body,
        grid=(N // 2000,),
        in_specs=[
            pl.BlockSpec((2000, D), lambda i: (i, 0)),
            pl.BlockSpec((2, D), lambda i: (0, 0)),
            pl.BlockSpec((1, D), lambda i: (0, 0)),
            pl.BlockSpec((1, D), lambda i: (0, 0)),
        ],
        out_specs=pl.BlockSpec((2000, D), lambda i: (i, 0)),
        out_shape=jax.ShapeDtypeStruct((N, D), jnp.float32),
    )(h_pre, stats, g.reshape(1, D), b.reshape(1, D))


def kernel(x, edge_index, batch, params):
    src = edge_index[0]
    dst = edge_index[1]

    h = _initial_linear(x, params["W_lin"], params["b_lin"])

    xs = []
    for p in params["layers"]:
        aggr = _segment_sum(h, src, dst, N)
        h_pre, stats = _layer_mlp(aggr, p)
        h = _batchnorm(h_pre, stats, p["bn_g"], p["bn_b"])
        xs.append(h)

    xcat = jnp.concatenate(xs, axis=1)
    pool = _segment_sum(xcat, jnp.arange(N, dtype=jnp.int32), batch, G)
    return (pool[:G], xcat)


# speedup vs baseline: 2.4332x; 1.2887x over previous
import functools

import jax
import jax.numpy as jnp
from jax import lax
from jax.experimental import pallas as pl
from jax.experimental.pallas import tpu as pltpu
from jax.experimental.pallas import tpu_sc as plsc

N = 10000
D = 128
G = 512

NC = 2
NS = 16
NW = NC * NS
LANES = 16
SCAN_CHUNK = 4096
FLUSH = 128
CAP = 2 * FLUSH


@functools.partial(jax.jit, static_argnames=("n_outer", "s_pad"))
def _sc_bucketize(src, dst, *, n_outer, s_pad):
    mesh = plsc.VectorSubcoreMesh(core_axis_name="c", subcore_axis_name="s")
    rpw = s_pad // NW
    e_pad = n_outer * SCAN_CHUNK

    @functools.partial(
        pl.kernel,
        out_type=[
            jax.ShapeDtypeStruct((NW, e_pad), jnp.int32),
            jax.ShapeDtypeStruct((NW, LANES), jnp.int32),
        ],
        mesh=mesh,
        compiler_params=pltpu.CompilerParams(needs_layout_passes=False),
        scratch_types=[
            pltpu.VMEM((SCAN_CHUNK,), jnp.int32),
            pltpu.VMEM((SCAN_CHUNK,), jnp.int32),
            pltpu.VMEM((CAP,), jnp.int32),
            pltpu.VMEM((LANES,), jnp.int32),
            pltpu.SemaphoreType.DMA,
        ],
    )
    def k(src_hbm, dst_hbm, bucket_hbm, counts_hbm,
          src_c, dst_c, pend, cbuf, sem):
        c = lax.axis_index("c")
        s = lax.axis_index("s")
        w = s * NC + c
        lo = w * rpw
        trash = jnp.int32(rpw * 16384)

        def group(g, carry):
            blk_base, cnt = carry
            off = blk_base + g * LANES
            dv = dst_c[pl.ds(off, LANES)]
            sv = src_c[pl.ds(off, LANES)]
            ld = dv - lo
            m = (ld >= 0) & (ld < rpw)
            cs = plsc.cumsum(m.astype(jnp.int32))
            pos = cs + (cnt - 1)
            plsc.store_scatter(pend, [pos], ld * 16384 + sv, mask=m)
            return (blk_base, cnt + cs[LANES - 1])

        def flush(nch):
            pltpu.async_copy(pend.at[pl.ds(0, FLUSH)],
                             bucket_hbm.at[w, pl.ds(nch * FLUSH, FLUSH)],
                             sem).wait()
            for t in range((CAP - FLUSH) // LANES):
                pend[pl.ds(t * LANES, LANES)] = (
                    pend[pl.ds(FLUSH + t * LANES, LANES)])

        def block(b, carry):
            cnt, nch = carry
            _, cnt = lax.fori_loop(0, FLUSH // LANES, group,
                                   (b * FLUSH, cnt), unroll=4)
            do = cnt >= FLUSH

            @pl.when(do)
            def _():
                flush(nch)

            return (jnp.where(do, cnt - FLUSH, cnt),
                    jnp.where(do, nch + 1, nch))

        def outer(o, carry):
            oe = lax.rem(o + w * (n_outer // NW), n_outer)
            base = oe * SCAN_CHUNK
            cp1 = pltpu.async_copy(src_hbm.at[pl.ds(base, SCAN_CHUNK)],
                                   src_c, sem)
            cp2 = pltpu.async_copy(dst_hbm.at[pl.ds(base, SCAN_CHUNK)],
                                   dst_c, sem)
            cp1.wait()
            cp2.wait()
            return lax.fori_loop(0, SCAN_CHUNK // FLUSH, block, carry)

        cnt, nch = lax.fori_loop(0, n_outer, outer,
                                 (jnp.int32(0), jnp.int32(0)))

        lane = lax.iota(jnp.int32, LANES)
        do_final = (cnt > 0) | (nch == 0)

        @pl.when(do_final)
        def _():
            for g in range(FLUSH // LANES):
                v = pend[pl.ds(g * LANES, LANES)]
                pend[pl.ds(g * LANES, LANES)] = jnp.where(
                    lane + (g * LANES) < cnt, v, trash)
            flush(nch)

        nch = jnp.where(do_final, nch + 1, nch)
        cbuf[...] = jnp.broadcast_to(nch, (LANES,))
        pltpu.sync_copy(cbuf, counts_hbm.at[w])

    return k(src, dst)


@functools.partial(jax.jit, static_argnames=("s_pad", "d"))
def _sc_bucket_segsum(table, bucket, counts, *, s_pad, d):
    mesh = plsc.VectorSubcoreMesh(core_axis_name="c", subcore_axis_name="s")
    rpw = s_pad // NW
    kd = d // LANES

    @functools.partial(
        pl.kernel,
        out_type=jax.ShapeDtypeStruct((s_pad, d), jnp.float32),
        mesh=mesh,
        compiler_params=pltpu.CompilerParams(needs_layout_passes=False),
        scratch_types=[
            pltpu.VMEM((FLUSH,), jnp.int32),
            pltpu.VMEM((FLUSH,), jnp.int32),
            pltpu.VMEM((FLUSH, d), jnp.float32),
            pltpu.VMEM((rpw + 1, d), jnp.float32),
            pltpu.VMEM((LANES,), jnp.int32),
            pltpu.SemaphoreType.DMA,
        ],
    )
    def k(table_hbm, bucket_hbm, counts_hbm, out_hbm,
          pk, sb, rows_v, acc, cbuf, sem):
        c = lax.axis_index("c")
        s = lax.axis_index("s")
        w = s * NC + c
        lo = w * rpw

        zf32 = jnp.zeros((LANES,), jnp.float32)

        def zrow(i, _):
            for kk in range(kd):
                acc[i, pl.ds(kk * LANES, LANES)] = zf32
            return ()

        lax.fori_loop(0, rpw + 1, zrow, ())

        pltpu.sync_copy(counts_hbm.at[w], cbuf)
        nch = cbuf[pl.ds(0, LANES)][0]

        def chunk_body(ch, _):
            pltpu.sync_copy(bucket_hbm.at[w, pl.ds(ch * FLUSH, FLUSH)], pk)
            for g in range(FLUSH // LANES):
                sb[pl.ds(g * LANES, LANES)] = (
                    pk[pl.ds(g * LANES, LANES)] & 16383)
            pltpu.async_copy(table_hbm.at[sb], rows_v, sem).wait()

            def grp_body(gj, _):
                ldvec = lax.shift_right_logical(
                    pk[pl.ds(gj * LANES, LANES)], 14)
                for jj in range(LANES):
                    ldst = ldvec[jj]
                    for kk in range(kd):
                        x = rows_v[gj * LANES + jj, pl.ds(kk * LANES, LANES)]
                        plsc.addupdate(
                            acc.at[ldst, pl.ds(kk * LANES, LANES)], x)
                return ()

            lax.fori_loop(0, FLUSH // LANES, grp_body, ())
            return ()

        lax.fori_loop(0, nch, chunk_body, ())

        pltpu.sync_copy(acc.at[pl.ds(0, rpw)], out_hbm.at[pl.ds(lo, rpw)])

    return k(table, bucket, counts)


def _prep_edges(src, dst, s_rows):
    e = src.shape[0]
    n_outer = -(-e // SCAN_CHUNK)
    e_pad = n_outer * SCAN_CHUNK
    s_pad = -(-s_rows // (NW * 8)) * (NW * 8)
    if e_pad != e:
        src = jnp.concatenate([src, jnp.zeros((e_pad - e,), jnp.int32)])
        dst = jnp.concatenate([dst, jnp.full((e_pad - e,), s_pad, jnp.int32)])
    bucket, counts = _sc_bucketize(src, dst, n_outer=n_outer, s_pad=s_pad)
    return bucket, counts, s_pad


BN_ROWS = 1000


def _lin_body(x_ref, w_ref, b_ref, o_ref):
    o_ref[...] = (
        jnp.dot(x_ref[...], w_ref[...], preferred_element_type=jnp.float32)
        + b_ref[...]
    )


def _initial_linear(x, w, b):
    return pl.pallas_call(
        _lin_body,
        grid=(N // 2000,),
        in_specs=[
            pl.BlockSpec((2000, D), lambda i: (i, 0)),
            pl.BlockSpec((D, D), lambda i: (0, 0)),
            pl.BlockSpec((1, D), lambda i: (0, 0)),
        ],
        out_specs=pl.BlockSpec((2000, D), lambda i: (i, 0)),
        out_shape=jax.ShapeDtypeStruct((N, D), jnp.float32),
    )(x, w, b.reshape(1, D))


def _layer_body(ag_ref, w1_ref, b1_ref, lng_ref, lnb_ref, w2_ref, b2_ref,
                hpre_ref, stats_ref, acc_ref):
    i = pl.program_id(0)
    a = ag_ref[...]
    t = jnp.dot(a, w1_ref[...], preferred_element_type=jnp.float32) + b1_ref[...]
    mu = jnp.mean(t, axis=1, keepdims=True)
    var = jnp.mean((t - mu) ** 2, axis=1, keepdims=True)
    t = (t - mu) * lax.rsqrt(var + 1e-5) * lng_ref[...] + lnb_ref[...]
    t = jnp.maximum(t, 0.0)
    u = jnp.dot(t, w2_ref[...], preferred_element_type=jnp.float32) + b2_ref[...]
    h = jnp.maximum(u + a, 0.0)
    hpre_ref[...] = h

    @pl.when(i == 0)
    def _():
        acc_ref[...] = jnp.zeros_like(acc_ref)

    acc_ref[0:1] += jnp.sum(h, axis=0, keepdims=True)
    acc_ref[1:2] += jnp.sum(h * h, axis=0, keepdims=True)

    @pl.when(i == pl.num_programs(0) - 1)
    def _():
        stats_ref[...] = acc_ref[...]


def _layer_mlp(aggr, p):
    return pl.pallas_call(
        _layer_body,
        grid=(N // BN_ROWS,),
        in_specs=[
            pl.BlockSpec((BN_ROWS, D), lambda i: (i, 0)),
            pl.BlockSpec((D, 2 * D), lambda i: (0, 0)),
            pl.BlockSpec((1, 2 * D), lambda i: (0, 0)),
            pl.BlockSpec((1, 2 * D), lambda i: (0, 0)),
            pl.BlockSpec((1, 2 * D), lambda i: (0, 0)),
            pl.BlockSpec((2 * D, D), lambda i: (0, 0)),
            pl.BlockSpec((1, D), lambda i: (0, 0)),
        ],
        out_specs=[
            pl.BlockSpec((BN_ROWS, D), lambda i: (i, 0)),
            pl.BlockSpec((2, D), lambda i: (0, 0)),
        ],
        out_shape=[
            jax.ShapeDtypeStruct((N, D), jnp.float32),
            jax.ShapeDtypeStruct((2, D), jnp.float32),
        ],
        scratch_shapes=[pltpu.VMEM((2, D), jnp.float32)],
    )(aggr, p["W1"], p["b1"].reshape(1, -1), p["ln_g"].reshape(1, -1),
      p["ln_b"].reshape(1, -1), p["W2"], p["b2"].reshape(1, -1))


def _bn_body(h_ref, stats_ref, g_ref, b_ref, o_ref):
    m = stats_ref[0:1] * (1.0 / N)
    v = stats_ref[1:2] * (1.0 / N) - m * m
    scale = lax.rsqrt(v + 1e-5) * g_ref[...]
    o_ref[...] = (h_ref[...] - m) * scale + b_ref[...]


def _batchnorm(h_pre, stats, g, b):
    return pl.pallas_call(
        _bn_body,
        grid=(N // 2000,),
        in_specs=[
            pl.BlockSpec((2000, D), lambda i: (i, 0)),
            pl.BlockSpec((2, D), lambda i: (0, 0)),
            pl.BlockSpec((1, D), lambda i: (0, 0)),
            pl.BlockSpec((1, D), lambda i: (0, 0)),
        ],
        out_specs=pl.BlockSpec((2000, D), lambda i: (i, 0)),
        out_shape=jax.ShapeDtypeStruct((N, D), jnp.float32),
    )(h_pre, stats, g.reshape(1, D), b.reshape(1, D))


def kernel(x, edge_index, batch, params):
    src = edge_index[0]
    dst = edge_index[1]

    h = _initial_linear(x, params["W_lin"], params["b_lin"])

    ebk, ecnt, s_pad_n = _prep_edges(src, dst, N)

    xs = []
    for p in params["layers"]:
        aggr = _sc_bucket_segsum(h, ebk, ecnt, s_pad=s_pad_n, d=D)
        h_pre, stats = _layer_mlp(aggr, p)
        h = _batchnorm(h_pre, stats, p["bn_g"], p["bn_b"])
        xs.append(h)

    xcat = jnp.concatenate(xs, axis=1)
    pbk, pcnt, s_pad_g = _prep_edges(jnp.arange(N, dtype=jnp.int32), batch, G)
    pool = _sc_bucket_segsum(xcat, pbk, pcnt, s_pad=s_pad_g, d=3 * D)
    return (pool[:G], xcat)


# speedup vs baseline: 2.5215x; 1.0363x over previous
import functools

import jax
import jax.numpy as jnp
from jax import lax
from jax.experimental import pallas as pl
from jax.experimental.pallas import tpu as pltpu
from jax.experimental.pallas import tpu_sc as plsc

N = 10000
D = 128
G = 512

NC = 2
NS = 16
NW = NC * NS
LANES = 16
SCAN_CHUNK = 4096
FLUSH = 128
CAP = 2 * FLUSH


@functools.partial(jax.jit, static_argnames=("n_outer", "s_pad"))
def _sc_bucketize(src, dst, *, n_outer, s_pad):
    mesh = plsc.VectorSubcoreMesh(core_axis_name="c", subcore_axis_name="s")
    rpw = s_pad // NW
    e_pad = n_outer * SCAN_CHUNK

    @functools.partial(
        pl.kernel,
        out_type=[
            jax.ShapeDtypeStruct((NW, e_pad), jnp.int32),
            jax.ShapeDtypeStruct((NW, LANES), jnp.int32),
        ],
        mesh=mesh,
        compiler_params=pltpu.CompilerParams(needs_layout_passes=False),
        scratch_types=[
            pltpu.VMEM((SCAN_CHUNK,), jnp.int32),
            pltpu.VMEM((SCAN_CHUNK,), jnp.int32),
            pltpu.VMEM((CAP,), jnp.int32),
            pltpu.VMEM((LANES,), jnp.int32),
            pltpu.SemaphoreType.DMA,
        ],
    )
    def k(src_hbm, dst_hbm, bucket_hbm, counts_hbm,
          src_c, dst_c, pend, cbuf, sem):
        c = lax.axis_index("c")
        s = lax.axis_index("s")
        w = s * NC + c
        lo = w * rpw
        trash = jnp.int32(rpw * 16384)

        def group(g, carry):
            blk_base, cnt = carry
            off = blk_base + g * LANES
            dv = dst_c[pl.ds(off, LANES)]
            sv = src_c[pl.ds(off, LANES)]
            ld = dv - lo
            m = (ld >= 0) & (ld < rpw)
            cs = plsc.cumsum(m.astype(jnp.int32))
            pos = cs + (cnt - 1)
            plsc.store_scatter(pend, [pos], ld * 16384 + sv, mask=m)
            return (blk_base, cnt + cs[LANES - 1])

        def flush(nch):
            pltpu.async_copy(pend.at[pl.ds(0, FLUSH)],
                             bucket_hbm.at[w, pl.ds(nch * FLUSH, FLUSH)],
                             sem).wait()
            for t in range((CAP - FLUSH) // LANES):
                pend[pl.ds(t * LANES, LANES)] = (
                    pend[pl.ds(FLUSH + t * LANES, LANES)])

        def block(b, carry):
            cnt, nch = carry
            _, cnt = lax.fori_loop(0, FLUSH // LANES, group,
                                   (b * FLUSH, cnt), unroll=4)
            do = cnt >= FLUSH

            @pl.when(do)
            def _():
                flush(nch)

            return (jnp.where(do, cnt - FLUSH, cnt),
                    jnp.where(do, nch + 1, nch))

        def outer(o, carry):
            oe = lax.rem(o + w * (n_outer // NW), n_outer)
            base = oe * SCAN_CHUNK
            cp1 = pltpu.async_copy(src_hbm.at[pl.ds(base, SCAN_CHUNK)],
                                   src_c, sem)
            cp2 = pltpu.async_copy(dst_hbm.at[pl.ds(base, SCAN_CHUNK)],
                                   dst_c, sem)
            cp1.wait()
            cp2.wait()
            return lax.fori_loop(0, SCAN_CHUNK // FLUSH, block, carry)

        cnt, nch = lax.fori_loop(0, n_outer, outer,
                                 (jnp.int32(0), jnp.int32(0)))

        lane = lax.iota(jnp.int32, LANES)
        do_final = (cnt > 0) | (nch == 0)

        @pl.when(do_final)
        def _():
            for g in range(FLUSH // LANES):
                v = pend[pl.ds(g * LANES, LANES)]
                pend[pl.ds(g * LANES, LANES)] = jnp.where(
                    lane + (g * LANES) < cnt, v, trash)
            flush(nch)

        nch = jnp.where(do_final, nch + 1, nch)

        odd = lax.rem(nch, 2) == 1

        @pl.when(odd)
        def _():
            tv = jnp.broadcast_to(trash, (LANES,))
            for g in range(FLUSH // LANES):
                pend[pl.ds(g * LANES, LANES)] = tv
            flush(nch)

        nch = jnp.where(odd, nch + 1, nch)
        cbuf[...] = jnp.broadcast_to(nch, (LANES,))
        pltpu.sync_copy(cbuf, counts_hbm.at[w])

    return k(src, dst)


@functools.partial(jax.jit, static_argnames=("s_pad", "d"))
def _sc_bucket_segsum(table, bucket, counts, *, s_pad, d):
    mesh = plsc.VectorSubcoreMesh(core_axis_name="c", subcore_axis_name="s")
    rpw = s_pad // NW
    kd = d // LANES

    @functools.partial(
        pl.kernel,
        out_type=jax.ShapeDtypeStruct((s_pad, d), jnp.float32),
        mesh=mesh,
        compiler_params=pltpu.CompilerParams(needs_layout_passes=False),
        scratch_types=[
            pltpu.VMEM((2, FLUSH), jnp.int32),
            pltpu.VMEM((2, FLUSH), jnp.int32),
            pltpu.VMEM((2, FLUSH), jnp.int32),
            pltpu.VMEM((2, FLUSH, d), jnp.float32),
            pltpu.VMEM((rpw + 1, d), jnp.float32),
            pltpu.VMEM((LANES,), jnp.int32),
            pltpu.SemaphoreType.DMA,
            pltpu.SemaphoreType.DMA,
            pltpu.SemaphoreType.DMA,
        ],
    )
    def k(table_hbm, bucket_hbm, counts_hbm, out_hbm,
          pk, sb, lb, rows_v, acc, cbuf, sem_s0, sem_s1, sem_g):
        c = lax.axis_index("c")
        s = lax.axis_index("s")
        w = s * NC + c
        lo = w * rpw
        sem_s = (sem_s0, sem_s1)

        zf32 = jnp.zeros((LANES,), jnp.float32)

        def zrow(i, _):
            for kk in range(kd):
                acc[i, pl.ds(kk * LANES, LANES)] = zf32
            return ()

        lax.fori_loop(0, rpw + 1, zrow, ())

        pltpu.sync_copy(counts_hbm.at[w], cbuf)
        nch = cbuf[pl.ds(0, LANES)][0]

        def stage(ch, b):
            cc = jnp.minimum(ch, nch - 1)
            return pltpu.async_copy(
                bucket_hbm.at[w, pl.ds(cc * FLUSH, FLUSH)], pk.at[b],
                sem_s[b])

        def unpack(b):
            for g in range(FLUSH // LANES):
                v = pk[b, pl.ds(g * LANES, LANES)]
                sb[b, pl.ds(g * LANES, LANES)] = v & 16383
                lb[b, pl.ds(g * LANES, LANES)] = lax.shift_right_logical(v, 14)

        def accumulate(b):
            def grp_body(gj, _):
                ldvec = lb[b, pl.ds(gj * LANES, LANES)]
                for jj in range(LANES):
                    ldst = ldvec[jj]
                    for kk in range(kd):
                        x = rows_v[b, gj * LANES + jj,
                                   pl.ds(kk * LANES, LANES)]
                        plsc.addupdate(
                            acc.at[ldst, pl.ds(kk * LANES, LANES)], x)
                return ()

            lax.fori_loop(0, FLUSH // LANES, grp_body, ())

        stage(jnp.int32(0), 0)
        stage(jnp.int32(1), 1)

        def stage_wait(b):
            pltpu.make_async_copy(bucket_hbm.at[w, pl.ds(0, FLUSH)],
                                  pk.at[b], sem_s[b]).wait()

        def pair_body(ch2, _):
            for b in (0, 1):
                ch = 2 * ch2 + b
                stage_wait(b)
                unpack(b)

                @pl.when(ch > 0)
                def _():
                    pltpu.make_async_copy(
                        table_hbm.at[sb.at[1 - b]], rows_v.at[1 - b],
                        sem_g).wait()

                pltpu.async_copy(table_hbm.at[sb.at[b]], rows_v.at[b], sem_g)
                stage(ch + 2, b)

                @pl.when(ch > 0)
                def _():
                    accumulate(1 - b)
            return ()

        lax.fori_loop(0, lax.shift_right_logical(nch, 1), pair_body, ())

        for b in (0, 1):
            pltpu.make_async_copy(
                bucket_hbm.at[w, pl.ds(0, FLUSH)], pk.at[b], sem_s[b]).wait()
        pltpu.make_async_copy(
            table_hbm.at[sb.at[1]], rows_v.at[1], sem_g).wait()
        accumulate(1)

        pltpu.sync_copy(acc.at[pl.ds(0, rpw)], out_hbm.at[pl.ds(lo, rpw)])

    return k(table, bucket, counts)


def _prep_edges(src, dst, s_rows):
    e = src.shape[0]
    n_outer = -(-e // SCAN_CHUNK)
    e_pad = n_outer * SCAN_CHUNK
    s_pad = -(-s_rows // (NW * 8)) * (NW * 8)
    if e_pad != e:
        src = jnp.concatenate([src, jnp.zeros((e_pad - e,), jnp.int32)])
        dst = jnp.concatenate([dst, jnp.full((e_pad - e,), s_pad, jnp.int32)])
    bucket, counts = _sc_bucketize(src, dst, n_outer=n_outer, s_pad=s_pad)
    return bucket, counts, s_pad


BN_ROWS = 1000


def _lin_body(x_ref, w_ref, b_ref, o_ref):
    o_ref[...] = (
        jnp.dot(x_ref[...], w_ref[...], preferred_element_type=jnp.float32)
        + b_ref[...]
    )


def _initial_linear(x, w, b):
    return pl.pallas_call(
        _lin_body,
        grid=(N // 2000,),
        in_specs=[
            pl.BlockSpec((2000, D), lambda i: (i, 0)),
            pl.BlockSpec((D, D), lambda i: (0, 0)),
            pl.BlockSpec((1, D), lambda i: (0, 0)),
        ],
        out_specs=pl.BlockSpec((2000, D), lambda i: (i, 0)),
        out_shape=jax.ShapeDtypeStruct((N, D), jnp.float32),
    )(x, w, b.reshape(1, D))


def _layer_body(ag_ref, w1_ref, b1_ref, lng_ref, lnb_ref, w2_ref, b2_ref,
                hpre_ref, stats_ref, acc_ref):
    i = pl.program_id(0)
    a = ag_ref[...]
    t = jnp.dot(a, w1_ref[...], preferred_element_type=jnp.float32) + b1_ref[...]
    mu = jnp.mean(t, axis=1, keepdims=True)
    var = jnp.mean((t - mu) ** 2, axis=1, keepdims=True)
    t = (t - mu) * lax.rsqrt(var + 1e-5) * lng_ref[...] + lnb_ref[...]
    t = jnp.maximum(t, 0.0)
    u = jnp.dot(t, w2_ref[...], preferred_element_type=jnp.float32) + b2_ref[...]
    h = jnp.maximum(u + a, 0.0)
    hpre_ref[...] = h

    @pl.when(i == 0)
    def _():
        acc_ref[...] = jnp.zeros_like(acc_ref)

    acc_ref[0:1] += jnp.sum(h, axis=0, keepdims=True)
    acc_ref[1:2] += jnp.sum(h * h, axis=0, keepdims=True)

    @pl.when(i == pl.num_programs(0) - 1)
    def _():
        stats_ref[...] = acc_ref[...]


def _layer_mlp(aggr, p):
    return pl.pallas_call(
        _layer_body,
        grid=(N // BN_ROWS,),
        in_specs=[
            pl.BlockSpec((BN_ROWS, D), lambda i: (i, 0)),
            pl.BlockSpec((D, 2 * D), lambda i: (0, 0)),
            pl.BlockSpec((1, 2 * D), lambda i: (0, 0)),
            pl.BlockSpec((1, 2 * D), lambda i: (0, 0)),
            pl.BlockSpec((1, 2 * D), lambda i: (0, 0)),
            pl.BlockSpec((2 * D, D), lambda i: (0, 0)),
            pl.BlockSpec((1, D), lambda i: (0, 0)),
        ],
        out_specs=[
            pl.BlockSpec((BN_ROWS, D), lambda i: (i, 0)),
            pl.BlockSpec((2, D), lambda i: (0, 0)),
        ],
        out_shape=[
            jax.ShapeDtypeStruct((N, D), jnp.float32),
            jax.ShapeDtypeStruct((2, D), jnp.float32),
        ],
        scratch_shapes=[pltpu.VMEM((2, D), jnp.float32)],
    )(aggr, p["W1"], p["b1"].reshape(1, -1), p["ln_g"].reshape(1, -1),
      p["ln_b"].reshape(1, -1), p["W2"], p["b2"].reshape(1, -1))


def _bn_body(h_ref, stats_ref, g_ref, b_ref, o_ref):
    m = stats_ref[0:1] * (1.0 / N)
    v = stats_ref[1:2] * (1.0 / N) - m * m
    scale = lax.rsqrt(v + 1e-5) * g_ref[...]
    o_ref[...] = (h_ref[...] - m) * scale + b_ref[...]


def _batchnorm(h_pre, stats, g, b):
    return pl.pallas_call(
        _bn_body,
        grid=(N // 2000,),
        in_specs=[
            pl.BlockSpec((2000, D), lambda i: (i, 0)),
            pl.BlockSpec((2, D), lambda i: (0, 0)),
            pl.BlockSpec((1, D), lambda i: (0, 0)),
            pl.BlockSpec((1, D), lambda i: (0, 0)),
        ],
        out_specs=pl.BlockSpec((2000, D), lambda i: (i, 0)),
        out_shape=jax.ShapeDtypeStruct((N, D), jnp.float32),
    )(h_pre, stats, g.reshape(1, D), b.reshape(1, D))


def kernel(x, edge_index, batch, params):
    src = edge_index[0]
    dst = edge_index[1]

    h = _initial_linear(x, params["W_lin"], params["b_lin"])

    ebk, ecnt, s_pad_n = _prep_edges(src, dst, N)

    xs = []
    for p in params["layers"]:
        aggr = _sc_bucket_segsum(h, ebk, ecnt, s_pad=s_pad_n, d=D)
        h_pre, stats = _layer_mlp(aggr, p)
        h = _batchnorm(h_pre, stats, p["bn_g"], p["bn_b"])
        xs.append(h)

    xcat = jnp.concatenate(xs, axis=1)
    pbk, pcnt, s_pad_g = _prep_edges(jnp.arange(N, dtype=jnp.int32), batch, G)
    pool = _sc_bucket_segsum(xcat, pbk, pcnt, s_pad=s_pad_g, d=3 * D)
    return (pool[:G], xcat)


# speedup vs baseline: 2.5396x; 1.0072x over previous
import functools

import jax
import jax.numpy as jnp
from jax import lax
from jax.experimental import pallas as pl
from jax.experimental.pallas import tpu as pltpu
from jax.experimental.pallas import tpu_sc as plsc

N = 10000
D = 128
G = 512

NC = 2
NS = 16
NW = NC * NS
LANES = 16
SCAN_CHUNK = 4096
FLUSH = 128
CAP = 2 * FLUSH


@functools.partial(jax.jit, static_argnames=("n_outer", "s_pad"))
def _sc_bucketize(src, dst, *, n_outer, s_pad):
    mesh = plsc.VectorSubcoreMesh(core_axis_name="c", subcore_axis_name="s")
    rpw = s_pad // NW
    e_pad = n_outer * SCAN_CHUNK

    @functools.partial(
        pl.kernel,
        out_type=[
            jax.ShapeDtypeStruct((NW, e_pad), jnp.int32),
            jax.ShapeDtypeStruct((NW, LANES), jnp.int32),
        ],
        mesh=mesh,
        compiler_params=pltpu.CompilerParams(needs_layout_passes=False),
        scratch_types=[
            pltpu.VMEM((SCAN_CHUNK,), jnp.int32),
            pltpu.VMEM((SCAN_CHUNK,), jnp.int32),
            pltpu.VMEM((CAP,), jnp.int32),
            pltpu.VMEM((LANES,), jnp.int32),
            pltpu.SemaphoreType.DMA,
        ],
    )
    def k(src_hbm, dst_hbm, bucket_hbm, counts_hbm,
          src_c, dst_c, pend, cbuf, sem):
        c = lax.axis_index("c")
        s = lax.axis_index("s")
        w = s * NC + c
        lo = w * rpw
        trash = jnp.int32(rpw * 16384)

        def group(g, carry):
            blk_base, cnt = carry
            off = blk_base + g * LANES
            dv = dst_c[pl.ds(off, LANES)]
            sv = src_c[pl.ds(off, LANES)]
            ld = dv - lo
            m = (ld >= 0) & (ld < rpw)
            cs = plsc.cumsum(m.astype(jnp.int32))
            pos = cs + (cnt - 1)
            plsc.store_scatter(pend, [pos], ld * 16384 + sv, mask=m)
            return (blk_base, cnt + cs[LANES - 1])

        def flush(nch):
            pltpu.async_copy(pend.at[pl.ds(0, FLUSH)],
                             bucket_hbm.at[w, pl.ds(nch * FLUSH, FLUSH)],
                             sem).wait()
            for t in range((CAP - FLUSH) // LANES):
                pend[pl.ds(t * LANES, LANES)] = (
                    pend[pl.ds(FLUSH + t * LANES, LANES)])

        def block(b, carry):
            cnt, nch = carry
            _, cnt = lax.fori_loop(0, FLUSH // LANES, group,
                                   (b * FLUSH, cnt), unroll=4)
            do = cnt >= FLUSH

            @pl.when(do)
            def _():
                flush(nch)

            return (jnp.where(do, cnt - FLUSH, cnt),
                    jnp.where(do, nch + 1, nch))

        def outer(o, carry):
            oe = lax.rem(o + w * (n_outer // NW), n_outer)
            base = oe * SCAN_CHUNK
            cp1 = pltpu.async_copy(src_hbm.at[pl.ds(base, SCAN_CHUNK)],
                                   src_c, sem)
            cp2 = pltpu.async_copy(dst_hbm.at[pl.ds(base, SCAN_CHUNK)],
                                   dst_c, sem)
            cp1.wait()
            cp2.wait()
            return lax.fori_loop(0, SCAN_CHUNK // FLUSH, block, carry)

        cnt, nch = lax.fori_loop(0, n_outer, outer,
                                 (jnp.int32(0), jnp.int32(0)))

        lane = lax.iota(jnp.int32, LANES)
        do_final = (cnt > 0) | (nch == 0)

        @pl.when(do_final)
        def _():
            for g in range(FLUSH // LANES):
                v = pend[pl.ds(g * LANES, LANES)]
                pend[pl.ds(g * LANES, LANES)] = jnp.where(
                    lane + (g * LANES) < cnt, v, trash)
            flush(nch)

        nch = jnp.where(do_final, nch + 1, nch)

        odd = lax.rem(nch, 2) == 1

        @pl.when(odd)
        def _():
            tv = jnp.broadcast_to(trash, (LANES,))
            for g in range(FLUSH // LANES):
                pend[pl.ds(g * LANES, LANES)] = tv
            flush(nch)

        nch = jnp.where(odd, nch + 1, nch)
        cbuf[...] = jnp.broadcast_to(nch, (LANES,))
        pltpu.sync_copy(cbuf, counts_hbm.at[w])

    return k(src, dst)


@functools.partial(jax.jit, static_argnames=("s_pad", "d"))
def _sc_bucket_segsum(table, bucket, counts, *, s_pad, d):
    mesh = plsc.VectorSubcoreMesh(core_axis_name="c", subcore_axis_name="s")
    rpw = s_pad // NW
    kd = d // LANES

    @functools.partial(
        pl.kernel,
        out_type=jax.ShapeDtypeStruct((s_pad, d), jnp.float32),
        mesh=mesh,
        compiler_params=pltpu.CompilerParams(needs_layout_passes=False),
        scratch_types=[
            pltpu.VMEM((2, FLUSH), jnp.int32),
            pltpu.VMEM((2, FLUSH), jnp.int32),
            pltpu.VMEM((2, FLUSH), jnp.int32),
            pltpu.VMEM((2, FLUSH, d), jnp.float32),
            pltpu.VMEM((rpw + 1, d), jnp.float32),
            pltpu.VMEM((LANES,), jnp.int32),
            pltpu.SemaphoreType.DMA,
            pltpu.SemaphoreType.DMA,
            pltpu.SemaphoreType.DMA,
        ],
    )
    def k(table_hbm, bucket_hbm, counts_hbm, out_hbm,
          pk, sb, lb, rows_v, acc, cbuf, sem_s0, sem_s1, sem_g):
        c = lax.axis_index("c")
        s = lax.axis_index("s")
        w = s * NC + c
        lo = w * rpw
        sem_s = (sem_s0, sem_s1)

        zf32 = jnp.zeros((LANES,), jnp.float32)

        def zrow(i, _):
            for kk in range(kd):
                acc[i, pl.ds(kk * LANES, LANES)] = zf32
            return ()

        lax.fori_loop(0, rpw + 1, zrow, ())

        pltpu.sync_copy(counts_hbm.at[w], cbuf)
        nch = cbuf[pl.ds(0, LANES)][0]

        def stage(ch, b):
            cc = jnp.minimum(ch, nch - 1)
            return pltpu.async_copy(
                bucket_hbm.at[w, pl.ds(cc * FLUSH, FLUSH)], pk.at[b],
                sem_s[b])

        def unpack(b):
            for g in range(FLUSH // LANES):
                v = pk[b, pl.ds(g * LANES, LANES)]
                sb[b, pl.ds(g * LANES, LANES)] = v & 16383
                lb[b, pl.ds(g * LANES, LANES)] = lax.shift_right_logical(v, 14)

        cols = [lax.iota(jnp.int32, LANES) + kk * LANES for kk in range(kd)]
        bidx = [jnp.broadcast_to(jnp.int32(jj), (LANES,))
                for jj in range(LANES)]

        def accumulate(b):
            def grp_body(gj, _):
                ldvec = lb[b, pl.ds(gj * LANES, LANES)]
                for jj in range(LANES):
                    ridx = ldvec.at[bidx[jj]].get(mode="promise_in_bounds")
                    for kk in range(kd):
                        x = rows_v[b, gj * LANES + jj,
                                   pl.ds(kk * LANES, LANES)]
                        plsc.addupdate_scatter(acc, [ridx, cols[kk]], x)
                return ()

            lax.fori_loop(0, FLUSH // LANES, grp_body, ())

        stage(jnp.int32(0), 0)
        stage(jnp.int32(1), 1)

        def stage_wait(b):
            pltpu.make_async_copy(bucket_hbm.at[w, pl.ds(0, FLUSH)],
                                  pk.at[b], sem_s[b]).wait()

        def pair_body(ch2, _):
            for b in (0, 1):
                ch = 2 * ch2 + b
                stage_wait(b)
                unpack(b)

                @pl.when(ch > 0)
                def _():
                    pltpu.make_async_copy(
                        table_hbm.at[sb.at[1 - b]], rows_v.at[1 - b],
                        sem_g).wait()

                pltpu.async_copy(table_hbm.at[sb.at[b]], rows_v.at[b], sem_g)
                stage(ch + 2, b)

                @pl.when(ch > 0)
                def _():
                    accumulate(1 - b)
            return ()

        lax.fori_loop(0, lax.shift_right_logical(nch, 1), pair_body, ())

        for b in (0, 1):
            pltpu.make_async_copy(
                bucket_hbm.at[w, pl.ds(0, FLUSH)], pk.at[b], sem_s[b]).wait()
        pltpu.make_async_copy(
            table_hbm.at[sb.at[1]], rows_v.at[1], sem_g).wait()
        accumulate(1)

        pltpu.sync_copy(acc.at[pl.ds(0, rpw)], out_hbm.at[pl.ds(lo, rpw)])

    return k(table, bucket, counts)


def _prep_edges(src, dst, s_rows):
    e = src.shape[0]
    n_outer = -(-e // SCAN_CHUNK)
    e_pad = n_outer * SCAN_CHUNK
    s_pad = -(-s_rows // (NW * 8)) * (NW * 8)
    if e_pad != e:
        src = jnp.concatenate([src, jnp.zeros((e_pad - e,), jnp.int32)])
        dst = jnp.concatenate([dst, jnp.full((e_pad - e,), s_pad, jnp.int32)])
    bucket, counts = _sc_bucketize(src, dst, n_outer=n_outer, s_pad=s_pad)
    return bucket, counts, s_pad


BN_ROWS = 1000


def _lin_body(x_ref, w_ref, b_ref, o_ref):
    o_ref[...] = (
        jnp.dot(x_ref[...], w_ref[...], preferred_element_type=jnp.float32)
        + b_ref[...]
    )


def _initial_linear(x, w, b):
    return pl.pallas_call(
        _lin_body,
        grid=(N // 2000,),
        in_specs=[
            pl.BlockSpec((2000, D), lambda i: (i, 0)),
            pl.BlockSpec((D, D), lambda i: (0, 0)),
            pl.BlockSpec((1, D), lambda i: (0, 0)),
        ],
        out_specs=pl.BlockSpec((2000, D), lambda i: (i, 0)),
        out_shape=jax.ShapeDtypeStruct((N, D), jnp.float32),
    )(x, w, b.reshape(1, D))


def _layer_body(ag_ref, w1_ref, b1_ref, lng_ref, lnb_ref, w2_ref, b2_ref,
                hpre_ref, stats_ref, acc_ref):
    i = pl.program_id(0)
    a = ag_ref[...]
    t = jnp.dot(a, w1_ref[...], preferred_element_type=jnp.float32) + b1_ref[...]
    mu = jnp.mean(t, axis=1, keepdims=True)
    var = jnp.mean((t - mu) ** 2, axis=1, keepdims=True)
    t = (t - mu) * lax.rsqrt(var + 1e-5) * lng_ref[...] + lnb_ref[...]
    t = jnp.maximum(t, 0.0)
    u = jnp.dot(t, w2_ref[...], preferred_element_type=jnp.float32) + b2_ref[...]
    h = jnp.maximum(u + a, 0.0)
    hpre_ref[...] = h

    @pl.when(i == 0)
    def _():
        acc_ref[...] = jnp.zeros_like(acc_ref)

    acc_ref[0:1] += jnp.sum(h, axis=0, keepdims=True)
    acc_ref[1:2] += jnp.sum(h * h, axis=0, keepdims=True)

    @pl.when(i == pl.num_programs(0) - 1)
    def _():
        stats_ref[...] = acc_ref[...]


def _layer_mlp(aggr, p):
    return pl.pallas_call(
        _layer_body,
        grid=(N // BN_ROWS,),
        in_specs=[
            pl.BlockSpec((BN_ROWS, D), lambda i: (i, 0)),
            pl.BlockSpec((D, 2 * D), lambda i: (0, 0)),
            pl.BlockSpec((1, 2 * D), lambda i: (0, 0)),
            pl.BlockSpec((1, 2 * D), lambda i: (0, 0)),
            pl.BlockSpec((1, 2 * D), lambda i: (0, 0)),
            pl.BlockSpec((2 * D, D), lambda i: (0, 0)),
            pl.BlockSpec((1, D), lambda i: (0, 0)),
        ],
        out_specs=[
            pl.BlockSpec((BN_ROWS, D), lambda i: (i, 0)),
            pl.BlockSpec((2, D), lambda i: (0, 0)),
        ],
        out_shape=[
            jax.ShapeDtypeStruct((N, D), jnp.float32),
            jax.ShapeDtypeStruct((2, D), jnp.float32),
        ],
        scratch_shapes=[pltpu.VMEM((2, D), jnp.float32)],
    )(aggr, p["W1"], p["b1"].reshape(1, -1), p["ln_g"].reshape(1, -1),
      p["ln_b"].reshape(1, -1), p["W2"], p["b2"].reshape(1, -1))


def _bn_body(h_ref, stats_ref, g_ref, b_ref, o_ref):
    m = stats_ref[0:1] * (1.0 / N)
    v = stats_ref[1:2] * (1.0 / N) - m * m
    scale = lax.rsqrt(v + 1e-5) * g_ref[...]
    o_ref[...] = (h_ref[...] - m) * scale + b_ref[...]


def _batchnorm(h_pre, stats, g, b):
    return pl.pallas_call(
        _bn_body,
        grid=(N // 2000,),
        in_specs=[
            pl.BlockSpec((2000, D), lambda i: (i, 0)),
            pl.BlockSpec((2, D), lambda i: (0, 0)),
            pl.BlockSpec((1, D), lambda i: (0, 0)),
            pl.BlockSpec((1, D), lambda i: (0, 0)),
        ],
        out_specs=pl.BlockSpec((2000, D), lambda i: (i, 0)),
        out_shape=jax.ShapeDtypeStruct((N, D), jnp.float32),
    )(h_pre, stats, g.reshape(1, D), b.reshape(1, D))


def kernel(x, edge_index, batch, params):
    src = edge_index[0]
    dst = edge_index[1]

    h = _initial_linear(x, params["W_lin"], params["b_lin"])

    ebk, ecnt, s_pad_n = _prep_edges(src, dst, N)

    xs = []
    for p in params["layers"]:
        aggr = _sc_bucket_segsum(h, ebk, ecnt, s_pad=s_pad_n, d=D)
        h_pre, stats = _layer_mlp(aggr, p)
        h = _batchnorm(h_pre, stats, p["bn_g"], p["bn_b"])
        xs.append(h)

    xcat = jnp.concatenate(xs, axis=1)
    pbk, pcnt, s_pad_g = _prep_edges(jnp.arange(N, dtype=jnp.int32), batch, G)
    pool = _sc_bucket_segsum(xcat, pbk, pcnt, s_pad=s_pad_g, d=3 * D)
    return (pool[:G], xcat)


# speedup vs baseline: 2.5404x; 1.0003x over previous
import functools

import jax
import jax.numpy as jnp
from jax import lax
from jax.experimental import pallas as pl
from jax.experimental.pallas import tpu as pltpu
from jax.experimental.pallas import tpu_sc as plsc

N = 10000
D = 128
G = 512

NC = 2
NS = 16
NW = NC * NS
LANES = 16
SCAN_CHUNK = 4096
FLUSH = 128
CAP = 2 * FLUSH


@functools.partial(jax.jit, static_argnames=("n_outer", "s_pad", "split"))
def _sc_bucketize(src, dst, *, n_outer, s_pad, split):
    mesh = plsc.VectorSubcoreMesh(core_axis_name="c", subcore_axis_name="s")
    rpw = s_pad // NS if split else s_pad // NW
    e_pad = n_outer * SCAN_CHUNK

    @functools.partial(
        pl.kernel,
        out_type=[
            jax.ShapeDtypeStruct((NW, e_pad), jnp.int32),
            jax.ShapeDtypeStruct((NW, LANES), jnp.int32),
        ],
        mesh=mesh,
        compiler_params=pltpu.CompilerParams(needs_layout_passes=False),
        scratch_types=[
            pltpu.VMEM((SCAN_CHUNK,), jnp.int32),
            pltpu.VMEM((SCAN_CHUNK,), jnp.int32),
            pltpu.VMEM((CAP,), jnp.int32),
            pltpu.VMEM((LANES,), jnp.int32),
            pltpu.SemaphoreType.DMA,
        ],
    )
    def k(src_hbm, dst_hbm, bucket_hbm, counts_hbm,
          src_c, dst_c, pend, cbuf, sem):
        c = lax.axis_index("c")
        s = lax.axis_index("s")
        w = s * NC + c
        lo = (s if split else w) * rpw
        src_lo = c * split
        trash = jnp.int32(rpw * 16384)

        def group(g, carry):
            blk_base, cnt = carry
            off = blk_base + g * LANES
            dv = dst_c[pl.ds(off, LANES)]
            sv = src_c[pl.ds(off, LANES)]
            ld = dv - lo
            m = (ld >= 0) & (ld < rpw)
            if split:
                m = m & (sv >= src_lo) & (sv < src_lo + split)
            cs = plsc.cumsum(m.astype(jnp.int32))
            pos = cs + (cnt - 1)
            plsc.store_scatter(pend, [pos], ld * 16384 + sv, mask=m)
            return (blk_base, cnt + cs[LANES - 1])

        def flush(nch):
            pltpu.async_copy(pend.at[pl.ds(0, FLUSH)],
                             bucket_hbm.at[w, pl.ds(nch * FLUSH, FLUSH)],
                             sem).wait()
            for t in range((CAP - FLUSH) // LANES):
                pend[pl.ds(t * LANES, LANES)] = (
                    pend[pl.ds(FLUSH + t * LANES, LANES)])

        def block(b, carry):
            cnt, nch = carry
            _, cnt = lax.fori_loop(0, FLUSH // LANES, group,
                                   (b * FLUSH, cnt), unroll=4)
            do = cnt >= FLUSH

            @pl.when(do)
            def _():
                flush(nch)

            return (jnp.where(do, cnt - FLUSH, cnt),
                    jnp.where(do, nch + 1, nch))

        def outer(o, carry):
            oe = lax.rem(o + w * (n_outer // NW), n_outer)
            base = oe * SCAN_CHUNK
            cp1 = pltpu.async_copy(src_hbm.at[pl.ds(base, SCAN_CHUNK)],
                                   src_c, sem)
            cp2 = pltpu.async_copy(dst_hbm.at[pl.ds(base, SCAN_CHUNK)],
                                   dst_c, sem)
            cp1.wait()
            cp2.wait()
            return lax.fori_loop(0, SCAN_CHUNK // FLUSH, block, carry)

        cnt, nch = lax.fori_loop(0, n_outer, outer,
                                 (jnp.int32(0), jnp.int32(0)))

        lane = lax.iota(jnp.int32, LANES)
        do_final = (cnt > 0) | (nch == 0)

        @pl.when(do_final)
        def _():
            for g in range(FLUSH // LANES):
                v = pend[pl.ds(g * LANES, LANES)]
                pend[pl.ds(g * LANES, LANES)] = jnp.where(
                    lane + (g * LANES) < cnt, v, trash)
            flush(nch)

        nch = jnp.where(do_final, nch + 1, nch)

        odd = lax.rem(nch, 2) == 1

        @pl.when(odd)
        def _():
            tv = jnp.broadcast_to(trash, (LANES,))
            for g in range(FLUSH // LANES):
                pend[pl.ds(g * LANES, LANES)] = tv
            flush(nch)

        nch = jnp.where(odd, nch + 1, nch)
        cbuf[...] = jnp.broadcast_to(nch, (LANES,))
        pltpu.sync_copy(cbuf, counts_hbm.at[w])

    return k(src, dst)


@functools.partial(jax.jit, static_argnames=("s_pad", "d", "stage_table"))
def _sc_bucket_segsum(table, bucket, counts, *, s_pad, d, stage_table):
    mesh = plsc.VectorSubcoreMesh(core_axis_name="c", subcore_axis_name="s")
    rpw = s_pad // NS if stage_table else s_pad // NW
    kd = d // LANES
    t_rows = table.shape[0]
    half = t_rows // 2
    tchunk = -(-(-(-half // NS)) // 8) * 8

    @functools.partial(
        pl.kernel,
        out_type=jax.ShapeDtypeStruct(
            (NC, s_pad, d) if stage_table else (s_pad, d), jnp.float32),
        mesh=mesh,
        compiler_params=pltpu.CompilerParams(needs_layout_passes=False),
        scratch_types=[
            pltpu.VMEM((2, FLUSH), jnp.int32),
            pltpu.VMEM((2, FLUSH), jnp.int32),
            pltpu.VMEM((2, FLUSH), jnp.int32),
            pltpu.VMEM((2, FLUSH, d), jnp.float32),
            pltpu.VMEM((rpw + 1, d), jnp.float32),
            pltpu.VMEM((LANES,), jnp.int32),
            pltpu.SemaphoreType.DMA,
            pltpu.SemaphoreType.DMA,
            pltpu.SemaphoreType.DMA,
        ] + ([pltpu.VMEM_SHARED((half, d), jnp.float32)]
             if stage_table else []),
    )
    def k(table_hbm, bucket_hbm, counts_hbm, out_hbm,
          pk, sb, lb, rows_v, acc, cbuf, sem_s0, sem_s1, sem_g, *tspm_opt):
        c = lax.axis_index("c")
        s = lax.axis_index("s")
        w = s * NC + c
        lo = (s if stage_table else w) * rpw
        sem_s = (sem_s0, sem_s1)

        if stage_table:
            tspm = tspm_opt[0]
            hbase = c * half
            for i in range(NS):
                off = i * tchunk
                ln = min(tchunk, half - off)
                if ln <= 0:
                    continue

                @pl.when(s == i)
                def _(off=off, ln=ln):
                    pltpu.sync_copy(table_hbm.at[pl.ds(hbase + off, ln)],
                                    tspm.at[pl.ds(off, ln)])
            plsc.subcore_barrier()
            gsrc = tspm
        else:
            gsrc = table_hbm

        zf32 = jnp.zeros((LANES,), jnp.float32)

        def zrow(i, _):
            for kk in range(kd):
                acc[i, pl.ds(kk * LANES, LANES)] = zf32
            return ()

        lax.fori_loop(0, rpw + 1, zrow, ())

        pltpu.sync_copy(counts_hbm.at[w], cbuf)
        nch = cbuf[pl.ds(0, LANES)][0]

        def stage(ch, b):
            cc = jnp.minimum(ch, nch - 1)
            return pltpu.async_copy(
                bucket_hbm.at[w, pl.ds(cc * FLUSH, FLUSH)], pk.at[b],
                sem_s[b])

        def unpack(b):
            for g in range(FLUSH // LANES):
                v = pk[b, pl.ds(g * LANES, LANES)]
                sv = v & 16383
                if stage_table:
                    sv = sv - (c * half)
                sb[b, pl.ds(g * LANES, LANES)] = sv
                lb[b, pl.ds(g * LANES, LANES)] = lax.shift_right_logical(v, 14)

        cols = [lax.iota(jnp.int32, LANES) + kk * LANES for kk in range(kd)]
        bidx = [jnp.broadcast_to(jnp.int32(jj), (LANES,))
                for jj in range(LANES)]

        def accumulate(b):
            def grp_body(gj, _):
                ldvec = lb[b, pl.ds(gj * LANES, LANES)]
                for jj in range(LANES):
                    ridx = ldvec.at[bidx[jj]].get(mode="promise_in_bounds")
                    for kk in range(kd):
                        x = rows_v[b, gj * LANES + jj,
                                   pl.ds(kk * LANES, LANES)]
                        plsc.addupdate_scatter(acc, [ridx, cols[kk]], x)
                return ()

            lax.fori_loop(0, FLUSH // LANES, grp_body, ())

        stage(jnp.int32(0), 0)
        stage(jnp.int32(1), 1)

        def stage_wait(b):
            pltpu.make_async_copy(bucket_hbm.at[w, pl.ds(0, FLUSH)],
                                  pk.at[b], sem_s[b]).wait()

        def pair_body(ch2, _):
            for b in (0, 1):
                ch = 2 * ch2 + b
                stage_wait(b)
                unpack(b)

                @pl.when(ch > 0)
                def _():
                    pltpu.make_async_copy(
                        gsrc.at[sb.at[1 - b]], rows_v.at[1 - b],
                        sem_g).wait()

                pltpu.async_copy(gsrc.at[sb.at[b]], rows_v.at[b], sem_g)
                stage(ch + 2, b)

                @pl.when(ch > 0)
                def _():
                    accumulate(1 - b)
            return ()

        lax.fori_loop(0, lax.shift_right_logical(nch, 1), pair_body, ())

        for b in (0, 1):
            pltpu.make_async_copy(
                bucket_hbm.at[w, pl.ds(0, FLUSH)], pk.at[b], sem_s[b]).wait()
        pltpu.make_async_copy(
            gsrc.at[sb.at[1]], rows_v.at[1], sem_g).wait()
        accumulate(1)

        if stage_table:
            pltpu.sync_copy(acc.at[pl.ds(0, rpw)],
                            out_hbm.at[c, pl.ds(lo, rpw)])
        else:
            pltpu.sync_copy(acc.at[pl.ds(0, rpw)], out_hbm.at[pl.ds(lo, rpw)])

    return k(table, bucket, counts)


def _prep_edges(src, dst, s_rows, split):
    e = src.shape[0]
    n_outer = -(-e // SCAN_CHUNK)
    e_pad = n_outer * SCAN_CHUNK
    s_pad = -(-s_rows // (NW * 8)) * (NW * 8)
    if e_pad != e:
        src = jnp.concatenate([src, jnp.zeros((e_pad - e,), jnp.int32)])
        dst = jnp.concatenate([dst, jnp.full((e_pad - e,), s_pad, jnp.int32)])
    bucket, counts = _sc_bucketize(src, dst, n_outer=n_outer, s_pad=s_pad,
                                   split=split)
    return bucket, counts, s_pad


BN_ROWS = 1000


def _lin_body(x_ref, w_ref, b_ref, o_ref):
    o_ref[...] = (
        jnp.dot(x_ref[...], w_ref[...], preferred_element_type=jnp.float32)
        + b_ref[...]
    )


def _initial_linear(x, w, b):
    return pl.pallas_call(
        _lin_body,
        grid=(N // 2000,),
        in_specs=[
            pl.BlockSpec((2000, D), lambda i: (i, 0)),
            pl.BlockSpec((D, D), lambda i: (0, 0)),
            pl.BlockSpec((1, D), lambda i: (0, 0)),
        ],
        out_specs=pl.BlockSpec((2000, D), lambda i: (i, 0)),
        out_shape=jax.ShapeDtypeStruct((N, D), jnp.float32),
    )(x, w, b.reshape(1, D))


def _layer_body(ag_ref, w1_ref, b1_ref, lng_ref, lnb_ref, w2_ref, b2_ref,
                hpre_ref, stats_ref, acc_ref):
    i = pl.program_id(0)
    a = ag_ref[...]
    t = jnp.dot(a, w1_ref[...], preferred_element_type=jnp.float32) + b1_ref[...]
    mu = jnp.mean(t, axis=1, keepdims=True)
    var = jnp.mean((t - mu) ** 2, axis=1, keepdims=True)
    t = (t - mu) * lax.rsqrt(var + 1e-5) * lng_ref[...] + lnb_ref[...]
    t = jnp.maximum(t, 0.0)
    u = jnp.dot(t, w2_ref[...], preferred_element_type=jnp.float32) + b2_ref[...]
    h = jnp.maximum(u + a, 0.0)
    hpre_ref[...] = h

    @pl.when(i == 0)
    def _():
        acc_ref[...] = jnp.zeros_like(acc_ref)

    acc_ref[0:1] += jnp.sum(h, axis=0, keepdims=True)
    acc_ref[1:2] += jnp.sum(h * h, axis=0, keepdims=True)

    @pl.when(i == pl.num_programs(0) - 1)
    def _():
        stats_ref[...] = acc_ref[...]


def _layer_mlp(aggr, p):
    return pl.pallas_call(
        _layer_body,
        grid=(N // BN_ROWS,),
        in_specs=[
            pl.BlockSpec((BN_ROWS, D), lambda i: (i, 0)),
            pl.BlockSpec((D, 2 * D), lambda i: (0, 0)),
            pl.BlockSpec((1, 2 * D), lambda i: (0, 0)),
            pl.BlockSpec((1, 2 * D), lambda i: (0, 0)),
            pl.BlockSpec((1, 2 * D), lambda i: (0, 0)),
            pl.BlockSpec((2 * D, D), lambda i: (0, 0)),
            pl.BlockSpec((1, D), lambda i: (0, 0)),
        ],
        out_specs=[
            pl.BlockSpec((BN_ROWS, D), lambda i: (i, 0)),
            pl.BlockSpec((2, D), lambda i: (0, 0)),
        ],
        out_shape=[
            jax.ShapeDtypeStruct((N, D), jnp.float32),
            jax.ShapeDtypeStruct((2, D), jnp.float32),
        ],
        scratch_shapes=[pltpu.VMEM((2, D), jnp.float32)],
    )(aggr, p["W1"], p["b1"].reshape(1, -1), p["ln_g"].reshape(1, -1),
      p["ln_b"].reshape(1, -1), p["W2"], p["b2"].reshape(1, -1))


def _bn_body(h_ref, stats_ref, g_ref, b_ref, o_ref):
    m = stats_ref[0:1] * (1.0 / N)
    v = stats_ref[1:2] * (1.0 / N) - m * m
    scale = lax.rsqrt(v + 1e-5) * g_ref[...]
    o_ref[...] = (h_ref[...] - m) * scale + b_ref[...]


def _batchnorm(h_pre, stats, g, b):
    return pl.pallas_call(
        _bn_body,
        grid=(N // 2000,),
        in_specs=[
            pl.BlockSpec((2000, D), lambda i: (i, 0)),
            pl.BlockSpec((2, D), lambda i: (0, 0)),
            pl.BlockSpec((1, D), lambda i: (0, 0)),
            pl.BlockSpec((1, D), lambda i: (0, 0)),
        ],
        out_specs=pl.BlockSpec((2000, D), lambda i: (i, 0)),
        out_shape=jax.ShapeDtypeStruct((N, D), jnp.float32),
    )(h_pre, stats, g.reshape(1, D), b.reshape(1, D))


def kernel(x, edge_index, batch, params):
    src = edge_index[0]
    dst = edge_index[1]

    h = _initial_linear(x, params["W_lin"], params["b_lin"])

    ebk, ecnt, s_pad_n = _prep_edges(src, dst, N, split=0)

    xs = []
    for p in params["layers"]:
        aggr = _sc_bucket_segsum(h, ebk, ecnt, s_pad=s_pad_n, d=D,
                                 stage_table=False)
        h_pre, stats = _layer_mlp(aggr, p)
        h = _batchnorm(h_pre, stats, p["bn_g"], p["bn_b"])
        xs.append(h)

    xcat = jnp.concatenate(xs, axis=1)
    pbk, pcnt, s_pad_g = _prep_edges(jnp.arange(N, dtype=jnp.int32),
                                     batch, G, split=0)
    pool = _sc_bucket_segsum(xcat, pbk, pcnt, s_pad=s_pad_g, d=3 * D,
                             stage_table=False)
    return (pool[:G], xcat)


# speedup vs baseline: 2.5848x; 1.0175x over previous
import functools

import jax
import jax.numpy as jnp
from jax import lax
from jax.experimental import pallas as pl
from jax.experimental.pallas import tpu as pltpu
from jax.experimental.pallas import tpu_sc as plsc

N = 10000
D = 128
G = 512

NC = 2
NS = 16
NW = NC * NS
LANES = 16
SCAN_CHUNK = 8192
FLUSH = 128
CAP = 2 * FLUSH


@functools.partial(jax.jit, static_argnames=("n_outer", "s_pad", "split"))
def _sc_bucketize(src, dst, *, n_outer, s_pad, split):
    mesh = plsc.VectorSubcoreMesh(core_axis_name="c", subcore_axis_name="s")
    rpw = s_pad // NS if split else s_pad // NW
    e_pad = n_outer * SCAN_CHUNK

    @functools.partial(
        pl.kernel,
        out_type=[
            jax.ShapeDtypeStruct((NW, e_pad), jnp.int32),
            jax.ShapeDtypeStruct((NW, LANES), jnp.int32),
        ],
        mesh=mesh,
        compiler_params=pltpu.CompilerParams(needs_layout_passes=False),
        scratch_types=[
            pltpu.VMEM((SCAN_CHUNK,), jnp.int32),
            pltpu.VMEM((SCAN_CHUNK,), jnp.int32),
            pltpu.VMEM((CAP,), jnp.int32),
            pltpu.VMEM((LANES,), jnp.int32),
            pltpu.SemaphoreType.DMA,
        ],
    )
    def k(src_hbm, dst_hbm, bucket_hbm, counts_hbm,
          src_c, dst_c, pend, cbuf, sem):
        c = lax.axis_index("c")
        s = lax.axis_index("s")
        w = s * NC + c
        lo = (s if split else w) * rpw
        src_lo = c * split
        trash = jnp.int32(rpw * 16384)

        def group(g, carry):
            blk_base, cnt = carry
            off = blk_base + g * LANES
            dv = dst_c[pl.ds(off, LANES)]
            sv = src_c[pl.ds(off, LANES)]
            ld = dv - lo
            m = (ld >= 0) & (ld < rpw)
            if split:
                m = m & (sv >= src_lo) & (sv < src_lo + split)
            cs = plsc.cumsum(m.astype(jnp.int32))
            pos = cs + (cnt - 1)
            plsc.store_scatter(pend, [pos], ld * 16384 + sv, mask=m)
            return (blk_base, cnt + cs[LANES - 1])

        def flush(nch):
            pltpu.async_copy(pend.at[pl.ds(0, FLUSH)],
                             bucket_hbm.at[w, pl.ds(nch * FLUSH, FLUSH)],
                             sem).wait()
            for t in range((CAP - FLUSH) // LANES):
                pend[pl.ds(t * LANES, LANES)] = (
                    pend[pl.ds(FLUSH + t * LANES, LANES)])

        def block(b, carry):
            cnt, nch = carry
            _, cnt = lax.fori_loop(0, FLUSH // LANES, group,
                                   (b * FLUSH, cnt), unroll=8)
            do = cnt >= FLUSH

            @pl.when(do)
            def _():
                flush(nch)

            return (jnp.where(do, cnt - FLUSH, cnt),
                    jnp.where(do, nch + 1, nch))

        def outer(o, carry):
            oe = lax.rem(o + w * (n_outer // NW), n_outer)
            base = oe * SCAN_CHUNK
            cp1 = pltpu.async_copy(src_hbm.at[pl.ds(base, SCAN_CHUNK)],
                                   src_c, sem)
            cp2 = pltpu.async_copy(dst_hbm.at[pl.ds(base, SCAN_CHUNK)],
                                   dst_c, sem)
            cp1.wait()
            cp2.wait()
            return lax.fori_loop(0, SCAN_CHUNK // FLUSH, block, carry)

        cnt, nch = lax.fori_loop(0, n_outer, outer,
                                 (jnp.int32(0), jnp.int32(0)))

        lane = lax.iota(jnp.int32, LANES)
        do_final = (cnt > 0) | (nch == 0)

        @pl.when(do_final)
        def _():
            for g in range(FLUSH // LANES):
                v = pend[pl.ds(g * LANES, LANES)]
                pend[pl.ds(g * LANES, LANES)] = jnp.where(
                    lane + (g * LANES) < cnt, v, trash)
            flush(nch)

        nch = jnp.where(do_final, nch + 1, nch)

        odd = lax.rem(nch, 2) == 1

        @pl.when(odd)
        def _():
            tv = jnp.broadcast_to(trash, (LANES,))
            for g in range(FLUSH // LANES):
                pend[pl.ds(g * LANES, LANES)] = tv
            flush(nch)

        nch = jnp.where(odd, nch + 1, nch)
        cbuf[...] = jnp.broadcast_to(nch, (LANES,))
        pltpu.sync_copy(cbuf, counts_hbm.at[w])

    return k(src, dst)


@functools.partial(jax.jit, static_argnames=("s_pad", "d", "stage_table"))
def _sc_bucket_segsum(table, bucket, counts, *, s_pad, d, stage_table):
    mesh = plsc.VectorSubcoreMesh(core_axis_name="c", subcore_axis_name="s")
    rpw = s_pad // NS if stage_table else s_pad // NW
    kd = d // LANES
    t_rows = table.shape[0]
    half = t_rows // 2
    tchunk = -(-(-(-half // NS)) // 8) * 8

    @functools.partial(
        pl.kernel,
        out_type=jax.ShapeDtypeStruct(
            (NC, s_pad, d) if stage_table else (s_pad, d), jnp.float32),
        mesh=mesh,
        compiler_params=pltpu.CompilerParams(needs_layout_passes=False),
        scratch_types=[
            pltpu.VMEM((2, FLUSH), jnp.int32),
            pltpu.VMEM((2, FLUSH), jnp.int32),
            pltpu.VMEM((2, FLUSH), jnp.int32),
            pltpu.VMEM((2, FLUSH, d), jnp.float32),
            pltpu.VMEM((rpw + 1, d), jnp.float32),
            pltpu.VMEM((LANES,), jnp.int32),
            pltpu.SemaphoreType.DMA,
            pltpu.SemaphoreType.DMA,
            pltpu.SemaphoreType.DMA,
        ] + ([pltpu.VMEM_SHARED((half, d), jnp.float32)]
             if stage_table else []),
    )
    def k(table_hbm, bucket_hbm, counts_hbm, out_hbm,
          pk, sb, lb, rows_v, acc, cbuf, sem_s0, sem_s1, sem_g, *tspm_opt):
        c = lax.axis_index("c")
        s = lax.axis_index("s")
        w = s * NC + c
        lo = (s if stage_table else w) * rpw
        sem_s = (sem_s0, sem_s1)

        if stage_table:
            tspm = tspm_opt[0]
            hbase = c * half
            for i in range(NS):
                off = i * tchunk
                ln = min(tchunk, half - off)
                if ln <= 0:
                    continue

                @pl.when(s == i)
                def _(off=off, ln=ln):
                    pltpu.sync_copy(table_hbm.at[pl.ds(hbase + off, ln)],
                                    tspm.at[pl.ds(off, ln)])
            plsc.subcore_barrier()
            gsrc = tspm
        else:
            gsrc = table_hbm

        zf32 = jnp.zeros((LANES,), jnp.float32)

        def zrow(i, _):
            for kk in range(kd):
                acc[i, pl.ds(kk * LANES, LANES)] = zf32
            return ()

        lax.fori_loop(0, rpw + 1, zrow, ())

        pltpu.sync_copy(counts_hbm.at[w], cbuf)
        nch = cbuf[pl.ds(0, LANES)][0]

        def stage(ch, b):
            cc = jnp.minimum(ch, nch - 1)
            return pltpu.async_copy(
                bucket_hbm.at[w, pl.ds(cc * FLUSH, FLUSH)], pk.at[b],
                sem_s[b])

        def unpack(b):
            for g in range(FLUSH // LANES):
                v = pk[b, pl.ds(g * LANES, LANES)]
                sv = v & 16383
                if stage_table:
                    sv = sv - (c * half)
                sb[b, pl.ds(g * LANES, LANES)] = sv
                lb[b, pl.ds(g * LANES, LANES)] = lax.shift_right_logical(v, 14)

        cols = [lax.iota(jnp.int32, LANES) + kk * LANES for kk in range(kd)]
        bidx = [jnp.broadcast_to(jnp.int32(jj), (LANES,))
                for jj in range(LANES)]

        def accumulate(b):
            def grp_body(gj, _):
                ldvec = lb[b, pl.ds(gj * LANES, LANES)]
                for jj in range(LANES):
                    ridx = ldvec.at[bidx[jj]].get(mode="promise_in_bounds")
                    for kk in range(kd):
                        x = rows_v[b, gj * LANES + jj,
                                   pl.ds(kk * LANES, LANES)]
                        plsc.addupdate_scatter(acc, [ridx, cols[kk]], x)
                return ()

            lax.fori_loop(0, FLUSH // LANES, grp_body, ())

        stage(jnp.int32(0), 0)
        stage(jnp.int32(1), 1)

        def stage_wait(b):
            pltpu.make_async_copy(bucket_hbm.at[w, pl.ds(0, FLUSH)],
                                  pk.at[b], sem_s[b]).wait()

        def pair_body(ch2, _):
            for b in (0, 1):
                ch = 2 * ch2 + b
                stage_wait(b)
                unpack(b)

                @pl.when(ch > 0)
                def _():
                    pltpu.make_async_copy(
                        gsrc.at[sb.at[1 - b]], rows_v.at[1 - b],
                        sem_g).wait()

                pltpu.async_copy(gsrc.at[sb.at[b]], rows_v.at[b], sem_g)
                stage(ch + 2, b)

                @pl.when(ch > 0)
                def _():
                    accumulate(1 - b)
            return ()

        lax.fori_loop(0, lax.shift_right_logical(nch, 1), pair_body, ())

        for b in (0, 1):
            pltpu.make_async_copy(
                bucket_hbm.at[w, pl.ds(0, FLUSH)], pk.at[b], sem_s[b]).wait()
        pltpu.make_async_copy(
            gsrc.at[sb.at[1]], rows_v.at[1], sem_g).wait()
        accumulate(1)

        if stage_table:
            pltpu.sync_copy(acc.at[pl.ds(0, rpw)],
                            out_hbm.at[c, pl.ds(lo, rpw)])
        else:
            pltpu.sync_copy(acc.at[pl.ds(0, rpw)], out_hbm.at[pl.ds(lo, rpw)])

    return k(table, bucket, counts)


def _prep_edges(src, dst, s_rows, split):
    e = src.shape[0]
    n_outer = -(-e // SCAN_CHUNK)
    e_pad = n_outer * SCAN_CHUNK
    s_pad = -(-s_rows // (NW * 8)) * (NW * 8)
    if e_pad != e:
        src = jnp.concatenate([src, jnp.zeros((e_pad - e,), jnp.int32)])
        dst = jnp.concatenate([dst, jnp.full((e_pad - e,), s_pad, jnp.int32)])
    bucket, counts = _sc_bucketize(src, dst, n_outer=n_outer, s_pad=s_pad,
                                   split=split)
    return bucket, counts, s_pad


BN_ROWS = 1000


def _lin_body(x_ref, w_ref, b_ref, o_ref):
    o_ref[...] = (
        jnp.dot(x_ref[...], w_ref[...], preferred_element_type=jnp.float32)
        + b_ref[...]
    )


def _initial_linear(x, w, b):
    return pl.pallas_call(
        _lin_body,
        grid=(N // 2000,),
        in_specs=[
            pl.BlockSpec((2000, D), lambda i: (i, 0)),
            pl.BlockSpec((D, D), lambda i: (0, 0)),
            pl.BlockSpec((1, D), lambda i: (0, 0)),
        ],
        out_specs=pl.BlockSpec((2000, D), lambda i: (i, 0)),
        out_shape=jax.ShapeDtypeStruct((N, D), jnp.float32),
    )(x, w, b.reshape(1, D))


def _layer_body(ag_ref, w1_ref, b1_ref, lng_ref, lnb_ref, w2_ref, b2_ref,
                hpre_ref, stats_ref, acc_ref):
    i = pl.program_id(0)
    a = ag_ref[...]
    t = jnp.dot(a, w1_ref[...], preferred_element_type=jnp.float32) + b1_ref[...]
    mu = jnp.mean(t, axis=1, keepdims=True)
    var = jnp.mean((t - mu) ** 2, axis=1, keepdims=True)
    t = (t - mu) * lax.rsqrt(var + 1e-5) * lng_ref[...] + lnb_ref[...]
    t = jnp.maximum(t, 0.0)
    u = jnp.dot(t, w2_ref[...], preferred_element_type=jnp.float32) + b2_ref[...]
    h = jnp.maximum(u + a, 0.0)
    hpre_ref[...] = h

    @pl.when(i == 0)
    def _():
        acc_ref[...] = jnp.zeros_like(acc_ref)

    acc_ref[0:1] += jnp.sum(h, axis=0, keepdims=True)
    acc_ref[1:2] += jnp.sum(h * h, axis=0, keepdims=True)

    @pl.when(i == pl.num_programs(0) - 1)
    def _():
        stats_ref[...] = acc_ref[...]


def _layer_mlp(aggr, p):
    return pl.pallas_call(
        _layer_body,
        grid=(N // BN_ROWS,),
        in_specs=[
            pl.BlockSpec((BN_ROWS, D), lambda i: (i, 0)),
            pl.BlockSpec((D, 2 * D), lambda i: (0, 0)),
            pl.BlockSpec((1, 2 * D), lambda i: (0, 0)),
            pl.BlockSpec((1, 2 * D), lambda i: (0, 0)),
            pl.BlockSpec((1, 2 * D), lambda i: (0, 0)),
            pl.BlockSpec((2 * D, D), lambda i: (0, 0)),
            pl.BlockSpec((1, D), lambda i: (0, 0)),
        ],
        out_specs=[
            pl.BlockSpec((BN_ROWS, D), lambda i: (i, 0)),
            pl.BlockSpec((2, D), lambda i: (0, 0)),
        ],
        out_shape=[
            jax.ShapeDtypeStruct((N, D), jnp.float32),
            jax.ShapeDtypeStruct((2, D), jnp.float32),
        ],
        scratch_shapes=[pltpu.VMEM((2, D), jnp.float32)],
    )(aggr, p["W1"], p["b1"].reshape(1, -1), p["ln_g"].reshape(1, -1),
      p["ln_b"].reshape(1, -1), p["W2"], p["b2"].reshape(1, -1))


def _bn_body(h_ref, stats_ref, g_ref, b_ref, o_ref):
    m = stats_ref[0:1] * (1.0 / N)
    v = stats_ref[1:2] * (1.0 / N) - m * m
    scale = lax.rsqrt(v + 1e-5) * g_ref[...]
    o_ref[...] = (h_ref[...] - m) * scale + b_ref[...]


def _batchnorm(h_pre, stats, g, b):
    return pl.pallas_call(
        _bn_body,
        grid=(N // 2000,),
        in_specs=[
            pl.BlockSpec((2000, D), lambda i: (i, 0)),
            pl.BlockSpec((2, D), lambda i: (0, 0)),
            pl.BlockSpec((1, D), lambda i: (0, 0)),
            pl.BlockSpec((1, D), lambda i: (0, 0)),
        ],
        out_specs=pl.BlockSpec((2000, D), lambda i: (i, 0)),
        out_shape=jax.ShapeDtypeStruct((N, D), jnp.float32),
    )(h_pre, stats, g.reshape(1, D), b.reshape(1, D))


def kernel(x, edge_index, batch, params):
    src = edge_index[0]
    dst = edge_index[1]

    h = _initial_linear(x, params["W_lin"], params["b_lin"])

    ebk, ecnt, s_pad_n = _prep_edges(src, dst, N, split=0)

    xs = []
    for p in params["layers"]:
        aggr = _sc_bucket_segsum(h, ebk, ecnt, s_pad=s_pad_n, d=D,
                                 stage_table=False)
        h_pre, stats = _layer_mlp(aggr, p)
        h = _batchnorm(h_pre, stats, p["bn_g"], p["bn_b"])
        xs.append(h)

    xcat = jnp.concatenate(xs, axis=1)
    pbk, pcnt, s_pad_g = _prep_edges(jnp.arange(N, dtype=jnp.int32),
                                     batch, G, split=0)
    pool = _sc_bucket_segsum(xcat, pbk, pcnt, s_pad=s_pad_g, d=3 * D,
                             stage_table=False)
    return (pool[:G], xcat)
